# baseline (speedup 1.0000x reference)
import jax
import jax.numpy as jnp
from jax import lax
from jax.experimental import pallas as pl
from jax.experimental.pallas import tpu as pltpu
from jax.experimental.pallas import tpu_sc as plsc

BATCH = 16384
DIM = 32
PACK = 128 // DIM
NUM_WORKERS = 32
B_PER_W = BATCH // NUM_WORKERS
CHUNK = 128
NCHUNK = B_PER_W // CHUNK
BLOCKS_PER_CHUNK = CHUNK // 16


def _body(sid_hbm, pid_hbm, s_packed, p_packed, out_hbm,
          sid_v, pid_v, sdiv_v, pdiv_v, srows, prows, out_v, sem_s, sem_p):
    num_cores = 2
    wid = lax.axis_index("s") * num_cores + lax.axis_index("c")
    base = wid * B_PER_W

    pltpu.sync_copy(sid_hbm.at[pl.ds(base, B_PER_W)], sid_v)
    pltpu.sync_copy(pid_hbm.at[pl.ds(base, B_PER_W)], pid_v)

    for i in range(B_PER_W // 16):
        sv = sid_v[pl.ds(i * 16, 16)]
        pv = pid_v[pl.ds(i * 16, 16)]
        c = i // BLOCKS_PER_CHUNK
        r = i % BLOCKS_PER_CHUNK
        sdiv_v[c, pl.ds(r * 16, 16)] = lax.shift_right_logical(sv, 2)
        pdiv_v[c, pl.ds(r * 16, 16)] = lax.shift_right_logical(pv, 2)

    lanes = lax.iota(jnp.int32, 16)

    for c in range(NCHUNK):
        cp_s = pltpu.async_copy(s_packed.at[sdiv_v.at[c]], srows, sem_s)
        cp_p = pltpu.async_copy(p_packed.at[pdiv_v.at[c]], prows, sem_p)
        cp_s.wait()
        cp_p.wait()

        def blk_body(lb, _):
            b = c * BLOCKS_PER_CHUNK + lb
            row_idx = lb * 16 + lanes
            sv = sid_v[pl.ds(b * 16, 16)]
            pv = pid_v[pl.ds(b * 16, 16)]
            scol0 = lax.shift_left(jnp.bitwise_and(sv, 3), 5)
            pcol0 = lax.shift_left(jnp.bitwise_and(pv, 3), 5)
            acc = jnp.zeros((16,), jnp.float32)
            for d in range(DIM):
                se = plsc.load_gather(srows, [row_idx, scol0 + d])
                pe = plsc.load_gather(prows, [row_idx, pcol0 + d])
                acc = acc + se * pe
            out_v[pl.ds(b * 16, 16)] = acc
            return ()

        lax.fori_loop(0, BLOCKS_PER_CHUNK, blk_body, ())

    pltpu.sync_copy(out_v, out_hbm.at[pl.ds(base, B_PER_W)])


@jax.jit
def kernel(sid, pid, scientist_table, paper_table):
    ns, _ = scientist_table.shape
    np_, _ = paper_table.shape
    s_packed = scientist_table.reshape(ns // PACK, 128)
    p_packed = paper_table.reshape(np_ // PACK, 128)
    mesh = plsc.VectorSubcoreMesh(core_axis_name="c", subcore_axis_name="s")
    run = pl.kernel(
        _body,
        out_type=jax.ShapeDtypeStruct((BATCH,), jnp.float32),
        mesh=mesh,
        scratch_types=[
            pltpu.VMEM((B_PER_W,), jnp.int32),
            pltpu.VMEM((B_PER_W,), jnp.int32),
            pltpu.VMEM((NCHUNK, CHUNK), jnp.int32),
            pltpu.VMEM((NCHUNK, CHUNK), jnp.int32),
            pltpu.VMEM((CHUNK, 128), jnp.float32),
            pltpu.VMEM((CHUNK, 128), jnp.float32),
            pltpu.VMEM((B_PER_W,), jnp.float32),
            pltpu.SemaphoreType.DMA,
            pltpu.SemaphoreType.DMA,
        ],
        compiler_params=pltpu.CompilerParams(
            needs_layout_passes=False, use_tc_tiling_on_sc=True),
    )
    return run(sid.astype(jnp.int32), pid.astype(jnp.int32),
               s_packed, p_packed)
